# baseline (speedup 1.0000x reference)
import functools

import jax
import jax.numpy as jnp
from jax import lax
from jax.experimental import pallas as pl
from jax.experimental.pallas import tpu as pltpu

ALPHA = 16.0


def _fused_body(ids_ref, x_ref, w_ref, b_ref, af_ref, b3_ref, o_ref, bf_ref,
                *, r, d_out):
    e = b3_ref.shape[0]

    @pl.when(pl.program_id(0) == 0)
    def _pack_b():
        for j in range(e):
            bf_ref[j * r:(j + 1) * r, :] = b3_ref[j]

    x = x_ref[...]
    base = jnp.dot(x, w_ref[...], preferred_element_type=jnp.float32)
    lr = jnp.dot(x, af_ref[...], preferred_element_type=jnp.float32)
    ids = ids_ref[...].reshape(lr.shape[0], 1)
    col_expert = lax.broadcasted_iota(jnp.int32, lr.shape, 1) // r
    mask = (col_expert == ids).astype(jnp.float32)
    delta = jnp.dot(lr * mask, bf_ref[...], preferred_element_type=jnp.float32)
    o_ref[...] = base + b_ref[...].reshape(1, d_out) + delta * (ALPHA / r)


def kernel(hidden_states, base_weight, base_bias, lora_a, lora_b, adapter_ids):
    n, d_in = hidden_states.shape
    d_out = base_weight.shape[1]
    e, _, r = lora_a.shape
    er = e * r

    a_flat = jnp.transpose(lora_a, (1, 0, 2)).reshape(d_in, er)

    tm = 1024
    grid = (n // tm,)

    body = functools.partial(_fused_body, r=r, d_out=d_out)
    return pl.pallas_call(
        body,
        grid=grid,
        in_specs=[
            pl.BlockSpec((tm,), lambda i: (i,)),
            pl.BlockSpec((tm, d_in), lambda i: (i, 0)),
            pl.BlockSpec((d_in, d_out), lambda i: (0, 0)),
            pl.BlockSpec((d_out,), lambda i: (0,)),
            pl.BlockSpec((d_in, er), lambda i: (0, 0)),
            pl.BlockSpec((e, r, d_out), lambda i: (0, 0, 0)),
        ],
        out_specs=pl.BlockSpec((tm, d_out), lambda i: (i, 0)),
        out_shape=jax.ShapeDtypeStruct((n, d_out), jnp.float32),
        scratch_shapes=[pltpu.VMEM((er, d_out), jnp.float32)],
        compiler_params=pltpu.CompilerParams(
            dimension_semantics=("arbitrary",),
        ),
    )(adapter_ids.astype(jnp.int32), hidden_states, base_weight, base_bias,
      a_flat, lora_b)


# speedup vs baseline: 1.0176x; 1.0176x over previous
import functools

import jax
import jax.numpy as jnp
from jax import lax
from jax.experimental import pallas as pl
from jax.experimental.pallas import tpu as pltpu

ALPHA = 16.0


def _fused_body(ids_ref, x_ref, wa_ref, b_ref, b3_ref, o_ref, bf_ref,
                *, r, d_out):
    e = b3_ref.shape[0]

    @pl.when(pl.program_id(0) == 0)
    def _pack_b():
        for j in range(e):
            bf_ref[j * r:(j + 1) * r, :] = b3_ref[j]

    x = x_ref[...]
    y = jnp.dot(x, wa_ref[...], preferred_element_type=jnp.float32)
    base = y[:, :d_out]
    lr = y[:, d_out:]
    ids = ids_ref[...].reshape(lr.shape[0], 1)
    col_expert = lax.broadcasted_iota(jnp.int32, lr.shape, 1) // r
    mask = (col_expert == ids).astype(jnp.float32)
    delta = jnp.dot(lr * mask, bf_ref[...], preferred_element_type=jnp.float32)
    o_ref[...] = base + b_ref[...].reshape(1, d_out) + delta * (ALPHA / r)


def kernel(hidden_states, base_weight, base_bias, lora_a, lora_b, adapter_ids):
    n, d_in = hidden_states.shape
    d_out = base_weight.shape[1]
    e, _, r = lora_a.shape
    er = e * r

    a_flat = jnp.transpose(lora_a, (1, 0, 2)).reshape(d_in, er)
    wa = jnp.concatenate([base_weight, a_flat], axis=1)

    tm = 1024
    grid = (n // tm,)

    body = functools.partial(_fused_body, r=r, d_out=d_out)
    return pl.pallas_call(
        body,
        grid=grid,
        in_specs=[
            pl.BlockSpec((tm,), lambda i: (i,)),
            pl.BlockSpec((tm, d_in), lambda i: (i, 0)),
            pl.BlockSpec((d_in, d_out + er), lambda i: (0, 0)),
            pl.BlockSpec((d_out,), lambda i: (0,)),
            pl.BlockSpec((e, r, d_out), lambda i: (0, 0, 0)),
        ],
        out_specs=pl.BlockSpec((tm, d_out), lambda i: (i, 0)),
        out_shape=jax.ShapeDtypeStruct((n, d_out), jnp.float32),
        scratch_shapes=[pltpu.VMEM((er, d_out), jnp.float32)],
        compiler_params=pltpu.CompilerParams(
            dimension_semantics=("arbitrary",),
        ),
    )(adapter_ids.astype(jnp.int32), hidden_states, wa, base_bias, lora_b)


# speedup vs baseline: 1.1017x; 1.0826x over previous
import functools

import jax
import jax.numpy as jnp
from jax import lax
from jax.experimental import pallas as pl
from jax.experimental.pallas import tpu as pltpu

ALPHA = 16.0


def _fused_body(ids_ref, x_ref, w_ref, b_ref, af_ref, b3_ref, o_ref,
                wa_ref, bf_ref, *, r, d_out):
    e = b3_ref.shape[0]
    er = e * r

    @pl.when(pl.program_id(0) == 0)
    def _pack():
        wa_ref[:, :d_out] = w_ref[...]
        wa_ref[:, d_out:] = af_ref[...]
        for j in range(e):
            bf_ref[j * r:(j + 1) * r, :] = b3_ref[j]

    x = x_ref[...]
    y = jnp.dot(x, wa_ref[...], preferred_element_type=jnp.float32)
    base = y[:, :d_out]
    lr = y[:, d_out:]
    ids = ids_ref[...].reshape(lr.shape[0], 1)
    col_expert = lax.broadcasted_iota(jnp.int32, lr.shape, 1) // r
    mask = (col_expert == ids).astype(jnp.float32)
    delta = jnp.dot(lr * mask, bf_ref[...], preferred_element_type=jnp.float32)
    o_ref[...] = base + b_ref[...].reshape(1, d_out) + delta * (ALPHA / r)


def kernel(hidden_states, base_weight, base_bias, lora_a, lora_b, adapter_ids):
    n, d_in = hidden_states.shape
    d_out = base_weight.shape[1]
    e, _, r = lora_a.shape
    er = e * r

    a_flat = jnp.transpose(lora_a, (1, 0, 2)).reshape(d_in, er)

    tm = 1024
    grid = (n // tm,)

    body = functools.partial(_fused_body, r=r, d_out=d_out)
    return pl.pallas_call(
        body,
        grid=grid,
        in_specs=[
            pl.BlockSpec((tm,), lambda i: (i,)),
            pl.BlockSpec((tm, d_in), lambda i: (i, 0)),
            pl.BlockSpec((d_in, d_out), lambda i: (0, 0)),
            pl.BlockSpec((d_out,), lambda i: (0,)),
            pl.BlockSpec((d_in, er), lambda i: (0, 0)),
            pl.BlockSpec((e, r, d_out), lambda i: (0, 0, 0)),
        ],
        out_specs=pl.BlockSpec((tm, d_out), lambda i: (i, 0)),
        out_shape=jax.ShapeDtypeStruct((n, d_out), jnp.float32),
        scratch_shapes=[
            pltpu.VMEM((d_in, d_out + er), jnp.float32),
            pltpu.VMEM((er, d_out), jnp.float32),
        ],
        compiler_params=pltpu.CompilerParams(
            dimension_semantics=("arbitrary",),
        ),
    )(adapter_ids.astype(jnp.int32), hidden_states, base_weight, base_bias,
      a_flat, lora_b)
